# Initial kernel scaffold; baseline (speedup 1.0000x reference)
import jax
import jax.numpy as jnp
from jax.experimental import pallas as pl


def kernel(input_ids, word_emb, pos_emb, type_emb, gamma, beta):
    raise NotImplementedError("write your pallas kernel here")


# speedup vs baseline: 1.1068x; 1.1068x over previous
import functools

import jax
import jax.numpy as jnp
from jax import lax
from jax.experimental import pallas as pl
from jax.experimental.pallas import tpu as pltpu
from jax.experimental.pallas import tpu_sc as plsc

LANES = 16
EPS = 1e-12


def _sc_embed_ln(ids2d, word_emb, pos_emb, type_emb, gamma, beta,
                 n_tok, seq_len, emb, n_workers):
    tpw = n_tok // n_workers
    idx_rows = tpw // 128
    n_chunks = emb // LANES

    mesh = plsc.VectorSubcoreMesh(core_axis_name="c", subcore_axis_name="s")
    num_cores = mesh.num_cores

    @functools.partial(
        pl.kernel,
        mesh=mesh,
        compiler_params=pltpu.CompilerParams(needs_layout_passes=False),
        out_type=jax.ShapeDtypeStruct((n_tok, emb), jnp.float32),
        scratch_types=[
            pltpu.VMEM((idx_rows, 128), jnp.int32),
            pltpu.VMEM((tpw, emb), jnp.float32),
            pltpu.VMEM((tpw, emb), jnp.float32),
            pltpu.VMEM((emb,), jnp.float32),
            pltpu.VMEM((emb,), jnp.float32),
            pltpu.VMEM((emb,), jnp.float32),
            pltpu.SemaphoreType.DMA,
        ],
    )
    def body(ids_hbm, word_hbm, pos_hbm, type_hbm, gam_hbm, bet_hbm,
             out_hbm, idx_v, rows_v, pos_v, type_v, gam_v, bet_v, sem):
        wid = lax.axis_index("s") * num_cores + lax.axis_index("c")
        base = wid * tpw
        srow = lax.rem(base, seq_len)

        pltpu.sync_copy(ids_hbm.at[pl.ds(wid * idx_rows, idx_rows)], idx_v)
        copies = [
            pltpu.async_copy(word_hbm.at[idx_v.at[j]],
                             rows_v.at[pl.ds(j * 128, 128)], sem)
            for j in range(idx_rows)
        ]
        pltpu.sync_copy(pos_hbm.at[pl.ds(srow, tpw)], pos_v)
        pltpu.sync_copy(type_hbm.at[0], type_v)
        pltpu.sync_copy(gam_hbm, gam_v)
        pltpu.sync_copy(bet_hbm, bet_v)
        for cp in copies:
            cp.wait()

        inv_n = 1.0 / emb

        def lane_sum(x):
            for sh in (8, 4, 2, 1):
                idx = lax.iota(jnp.int32, 16) ^ sh
                x = x + x.at[idx].get(mode="promise_in_bounds",
                                      unique_indices=True)
            return x

        def token_body(i, carry):
            s1 = jnp.zeros((LANES,), jnp.float32)
            s2 = jnp.zeros((LANES,), jnp.float32)
            for c in range(n_chunks):
                sl = pl.ds(c * LANES, LANES)
                x = rows_v[i, sl] + pos_v[i, sl] + type_v[sl]
                s1 = s1 + x
                s2 = s2 + x * x
                rows_v[i, sl] = x
            mean = lane_sum(s1) * inv_n
            msq = lane_sum(s2) * inv_n
            v = msq - mean * mean + EPS
            iv = plsc.bitcast(v, jnp.int32)
            seed = jnp.full((LANES,), 0x5F3759DF, jnp.int32)
            y = plsc.bitcast(seed - (iv >> 1), jnp.float32)
            for _ in range(3):
                y = y * (1.5 - 0.5 * v * y * y)
            for c in range(n_chunks):
                sl = pl.ds(c * LANES, LANES)
                x = rows_v[i, sl]
                rows_v[i, sl] = (x - mean) * y * gam_v[sl] + bet_v[sl]
            return carry

        lax.fori_loop(0, tpw, token_body, 0)
        pltpu.sync_copy(rows_v, out_hbm.at[pl.ds(base, tpw)])

    return body(ids2d, word_emb, pos_emb, type_emb, gamma, beta)


def kernel(input_ids, word_emb, pos_emb, type_emb, gamma, beta):
    b, s = input_ids.shape
    emb = word_emb.shape[1]
    n_tok = b * s
    n_workers = 32
    ids2d = input_ids.astype(jnp.int32).reshape(n_tok // 128, 128)
    out = _sc_embed_ln(ids2d, word_emb, pos_emb, type_emb, gamma, beta,
                       n_tok, s, emb, n_workers)
    return out.reshape(b, s, emb)


# speedup vs baseline: 1.7832x; 1.6111x over previous
import functools

import jax
import jax.numpy as jnp
from jax import lax
from jax.experimental import pallas as pl
from jax.experimental.pallas import tpu as pltpu
from jax.experimental.pallas import tpu_sc as plsc

LANES = 16
EPS = 1e-12


def _sc_embed_ln(ids2d, word_emb, pos_emb, type_emb, gamma, beta,
                 n_tok, seq_len, emb, n_workers):
    tpw = n_tok // n_workers
    idx_rows = tpw // 128
    n_chunks = emb // LANES

    mesh = plsc.VectorSubcoreMesh(core_axis_name="c", subcore_axis_name="s")
    num_cores = mesh.num_cores

    @functools.partial(
        pl.kernel,
        mesh=mesh,
        compiler_params=pltpu.CompilerParams(needs_layout_passes=False),
        out_type=jax.ShapeDtypeStruct((n_tok, emb), jnp.float32),
        scratch_types=[
            pltpu.VMEM((idx_rows, 128), jnp.int32),
            pltpu.VMEM((tpw, emb), jnp.float32),
            pltpu.VMEM((tpw, emb), jnp.float32),
            pltpu.VMEM((emb,), jnp.float32),
            pltpu.VMEM((emb,), jnp.float32),
            pltpu.VMEM((emb,), jnp.float32),
            pltpu.SemaphoreType.DMA,
        ],
    )
    def body(ids_hbm, word_hbm, pos_hbm, type_hbm, gam_hbm, bet_hbm,
             out_hbm, idx_v, rows_v, pos_v, type_v, gam_v, bet_v, sem):
        wid = lax.axis_index("s") * num_cores + lax.axis_index("c")
        base = wid * tpw
        srow = lax.rem(base, seq_len)

        pltpu.sync_copy(ids_hbm.at[pl.ds(wid * idx_rows, idx_rows)], idx_v)
        copies = [
            pltpu.async_copy(word_hbm.at[idx_v.at[j]],
                             rows_v.at[pl.ds(j * 128, 128)], sem)
            for j in range(idx_rows)
        ]
        pltpu.sync_copy(pos_hbm.at[pl.ds(srow, tpw)], pos_v)
        pltpu.sync_copy(type_hbm.at[0], type_v)
        pltpu.sync_copy(gam_hbm, gam_v)
        pltpu.sync_copy(bet_hbm, bet_v)
        for cp in copies:
            cp.wait()

        inv_n = 1.0 / emb

        def lane_sum(x):
            for sh in (8, 4, 2, 1):
                idx = lax.iota(jnp.int32, 16) ^ sh
                x = x + x.at[idx].get(mode="promise_in_bounds",
                                      unique_indices=True)
            return x

        ts = [type_v[pl.ds(c * LANES, LANES)] for c in range(n_chunks)]
        gs = [gam_v[pl.ds(c * LANES, LANES)] for c in range(n_chunks)]
        bs = [bet_v[pl.ds(c * LANES, LANES)] for c in range(n_chunks)]
        seed = jnp.full((LANES,), 0x5F3759DF, jnp.int32)

        @plsc.parallel_loop(0, tpw, unroll=4)
        def token_body(i):
            xs = []
            s1 = jnp.zeros((LANES,), jnp.float32)
            s2 = jnp.zeros((LANES,), jnp.float32)
            for c in range(n_chunks):
                sl = pl.ds(c * LANES, LANES)
                x = rows_v[i, sl] + pos_v[i, sl] + ts[c]
                xs.append(x)
                s1 = s1 + x
                s2 = s2 + x * x
            mean = lane_sum(s1) * inv_n
            msq = lane_sum(s2) * inv_n
            v = msq - mean * mean + EPS
            iv = plsc.bitcast(v, jnp.int32)
            y = plsc.bitcast(seed - (iv >> 1), jnp.float32)
            for _ in range(3):
                y = y * (1.5 - 0.5 * v * y * y)
            for c in range(n_chunks):
                rows_v[i, pl.ds(c * LANES, LANES)] = \
                    (xs[c] - mean) * y * gs[c] + bs[c]
        pltpu.sync_copy(rows_v, out_hbm.at[pl.ds(base, tpw)])

    return body(ids2d, word_emb, pos_emb, type_emb, gamma, beta)


def kernel(input_ids, word_emb, pos_emb, type_emb, gamma, beta):
    b, s = input_ids.shape
    emb = word_emb.shape[1]
    n_tok = b * s
    n_workers = 32
    ids2d = input_ids.astype(jnp.int32).reshape(n_tok // 128, 128)
    out = _sc_embed_ln(ids2d, word_emb, pos_emb, type_emb, gamma, beta,
                       n_tok, s, emb, n_workers)
    return out.reshape(b, s, emb)
